# initial kernel scaffold (unmeasured)
import jax
import jax.numpy as jnp
from jax import lax
from jax.experimental import pallas as pl
from jax.experimental.pallas import tpu as pltpu

N_DEV = 32
W_SLOTS = 3


def kernel(x, w_mat):
    m_per, k = x.shape
    n = w_mat.shape[1]
    n_per = n // N_DEV
    m = m_per * N_DEV

    def body(x_ref, w_hbm, out_ref, w_buf, send_buf, amax_buf,
             w_sems, send_sems, recv_sems, asend_sems, arecv_sems):
        my = lax.axis_index("i")

        def w_dma(t):
            j = (my + t) % N_DEV
            return pltpu.make_async_copy(
                w_hbm.at[:, pl.ds(j * n_per, n_per)],
                w_buf.at[t % W_SLOTS],
                w_sems.at[t % W_SLOTS],
            )

        w_dma(0).start()
        w_dma(1).start()

        data_rdmas = []
        amax = jnp.float32(0.0)
        for t in range(N_DEV):
            if t + 2 < N_DEV:
                w_dma(t + 2).start()
            w_dma(t).wait()
            y = jnp.maximum(
                jnp.dot(x_ref[:, :], w_buf[t % W_SLOTS],
                        preferred_element_type=jnp.float32),
                0.0,
            )
            amax = jnp.maximum(amax, jnp.max(y))
            if t == 0:
                out_ref[pl.ds(my * m_per, m_per), :] = y
            else:
                j = (my + t) % N_DEV
                send_buf[t, :, :] = y
                rdma = pltpu.make_async_remote_copy(
                    src_ref=send_buf.at[t],
                    dst_ref=out_ref.at[pl.ds(my * m_per, m_per)],
                    send_sem=send_sems.at[t],
                    recv_sem=recv_sems.at[t],
                    device_id=(j,),
                    device_id_type=pl.DeviceIdType.MESH,
                )
                rdma.start()
                data_rdmas.append(rdma)

        amax_buf[pl.ds(0, 1), :] = jnp.full((1, 128), amax, jnp.float32)
        amax_rdmas = []
        for t in range(1, N_DEV):
            j = (my + t) % N_DEV
            r = pltpu.make_async_remote_copy(
                src_ref=amax_buf.at[pl.ds(0, 1)],
                dst_ref=amax_buf.at[pl.ds(t, 1)],
                send_sem=asend_sems.at[t],
                recv_sem=arecv_sems.at[t],
                device_id=(j,),
                device_id_type=pl.DeviceIdType.MESH,
            )
            r.start()
            amax_rdmas.append(r)

        for r in data_rdmas:
            r.wait_recv()
        for r in amax_rdmas:
            r.wait_recv()

        gmax = jnp.max(amax_buf[:, :])
        scale = gmax / 448.0
        q = jnp.minimum(out_ref[:, :] / scale, 448.0)
        out_ref[:, :] = q.astype(jnp.float8_e4m3fn).astype(jnp.float32) * scale

        for r in data_rdmas:
            r.wait_send()
        for r in amax_rdmas:
            r.wait_send()

    return pl.pallas_call(
        body,
        out_shape=jax.ShapeDtypeStruct((m, n_per), jnp.float32),
        in_specs=[
            pl.BlockSpec(memory_space=pltpu.VMEM),
            pl.BlockSpec(memory_space=pltpu.ANY),
        ],
        out_specs=pl.BlockSpec(memory_space=pltpu.VMEM),
        scratch_shapes=[
            pltpu.VMEM((W_SLOTS, k, n_per), jnp.float32),
            pltpu.VMEM((N_DEV, m_per, n_per), jnp.float32),
            pltpu.VMEM((N_DEV, 128), jnp.float32),
            pltpu.SemaphoreType.DMA((W_SLOTS,)),
            pltpu.SemaphoreType.DMA((N_DEV,)),
            pltpu.SemaphoreType.DMA((N_DEV,)),
            pltpu.SemaphoreType.DMA((N_DEV,)),
            pltpu.SemaphoreType.DMA((N_DEV,)),
        ],
        compiler_params=pltpu.CompilerParams(collective_id=0),
    )(x, w_mat)


# baseline (device time: 81147 ns/iter reference)
import jax
import jax.numpy as jnp
from jax import lax
from jax.experimental import pallas as pl
from jax.experimental.pallas import tpu as pltpu

N_DEV = 32
W_SLOTS = 3


def kernel(x, w_mat):
    m_per, k = x.shape
    n = w_mat.shape[1]
    n_per = n // N_DEV
    m = m_per * N_DEV

    def body(x_ref, w_hbm, out_ref, w_buf, send_buf, amax_buf,
             w_sems, send_sems, recv_sems, asend_sems, arecv_sems):
        my = lax.axis_index("i")

        def w_dma(t):
            j = (my + t) % N_DEV
            return pltpu.make_async_copy(
                w_hbm.at[:, pl.ds(j * n_per, n_per)],
                w_buf.at[t % W_SLOTS],
                w_sems.at[t % W_SLOTS],
            )

        w_dma(0).start()
        w_dma(1).start()

        data_rdmas = []
        amax = jnp.float32(0.0)
        for t in range(N_DEV):
            if t + 2 < N_DEV:
                w_dma(t + 2).start()
            w_dma(t).wait()
            y = jnp.maximum(
                jnp.dot(x_ref[:, :], w_buf[t % W_SLOTS],
                        preferred_element_type=jnp.float32),
                0.0,
            )
            amax = jnp.maximum(amax, jnp.max(y))
            if t == 0:
                out_ref[pl.ds(my * m_per, m_per), :] = y
            else:
                j = (my + t) % N_DEV
                send_buf[t, :, :] = y
                rdma = pltpu.make_async_remote_copy(
                    src_ref=send_buf.at[t],
                    dst_ref=out_ref.at[pl.ds(my * m_per, m_per)],
                    send_sem=send_sems.at[t],
                    recv_sem=recv_sems.at[t],
                    device_id=(j,),
                    device_id_type=pl.DeviceIdType.MESH,
                )
                rdma.start()
                data_rdmas.append(rdma)

        amax_buf[pl.ds(0, 1), :] = jnp.full((1, 128), amax, jnp.float32)
        amax_rdmas = []
        for t in range(1, N_DEV):
            j = (my + t) % N_DEV
            r = pltpu.make_async_remote_copy(
                src_ref=amax_buf.at[pl.ds(0, 1)],
                dst_ref=amax_buf.at[pl.ds(t, 1)],
                send_sem=asend_sems.at[t],
                recv_sem=arecv_sems.at[t],
                device_id=(j,),
                device_id_type=pl.DeviceIdType.MESH,
            )
            r.start()
            amax_rdmas.append(r)

        for r in data_rdmas:
            r.wait_recv()
        for r in amax_rdmas:
            r.wait_recv()

        gmax = jnp.max(amax_buf[:, :])
        scale = gmax / 448.0
        q = jnp.minimum(out_ref[:, :] / scale, 448.0)
        out_ref[:, :] = q.astype(jnp.float8_e4m3fn).astype(jnp.float32) * scale

        for r in data_rdmas:
            r.wait_send()
        for r in amax_rdmas:
            r.wait_send()

    return pl.pallas_call(
        body,
        out_shape=jax.ShapeDtypeStruct((m, n_per), jnp.float32),
        in_specs=[
            pl.BlockSpec(memory_space=pltpu.VMEM),
            pl.BlockSpec(memory_space=pl.ANY),
        ],
        out_specs=pl.BlockSpec(memory_space=pltpu.VMEM),
        scratch_shapes=[
            pltpu.VMEM((W_SLOTS, k, n_per), jnp.float32),
            pltpu.VMEM((N_DEV, m_per, n_per), jnp.float32),
            pltpu.VMEM((N_DEV, 128), jnp.float32),
            pltpu.SemaphoreType.DMA((W_SLOTS,)),
            pltpu.SemaphoreType.DMA((N_DEV,)),
            pltpu.SemaphoreType.DMA((N_DEV,)),
            pltpu.SemaphoreType.DMA((N_DEV,)),
            pltpu.SemaphoreType.DMA((N_DEV,)),
        ],
    )(x, w_mat)
